# baseline (speedup 1.0000x reference)
import jax
import jax.numpy as jnp
from jax import lax
from jax.experimental import pallas as pl
from jax.experimental.pallas import tpu as pltpu
from jax.experimental.pallas import tpu_sc as plsc

HIDDEN = 2048
NUM_EXPERTS = 16
TOP_K = 2


def _logits_body(x_ref, w_ref, out_ref):
    out_ref[...] = lax.dot_general(
        x_ref[...],
        w_ref[...],
        dimension_numbers=(((1,), (1,)), ((), ())),
        preferred_element_type=jnp.float32,
    )


def _logits(xf, W):
    n = xf.shape[0]
    bt = 1024
    return pl.pallas_call(
        _logits_body,
        grid=(n // bt,),
        in_specs=[
            pl.BlockSpec((bt, HIDDEN), lambda i: (i, 0)),
            pl.BlockSpec((NUM_EXPERTS, HIDDEN), lambda i: (0, 0)),
        ],
        out_specs=pl.BlockSpec((bt, NUM_EXPERTS), lambda i: (i, 0)),
        out_shape=jax.ShapeDtypeStruct((n, NUM_EXPERTS), jnp.float32),
        compiler_params=pltpu.CompilerParams(vmem_limit_bytes=100 * 2**20),
    )(xf, W)


def _router(logits, n):
    info = plsc.get_sparse_core_info()
    nw = info.num_cores * info.num_subcores
    tpw = n // nw
    chunk = 128
    mesh = plsc.VectorSubcoreMesh(core_axis_name="c", subcore_axis_name="s")

    nchunks = tpw // chunk

    @pl.kernel(
        out_type=(
            jax.ShapeDtypeStruct((n, NUM_EXPERTS), jnp.float32),
            jax.ShapeDtypeStruct((n * TOP_K,), jnp.float32),
            jax.ShapeDtypeStruct((n * TOP_K,), jnp.int32),
        ),
        mesh=mesh,
        compiler_params=pltpu.CompilerParams(needs_layout_passes=False),
        scratch_types=[
            pltpu.VMEM((2, chunk, NUM_EXPERTS), jnp.float32),
            pltpu.VMEM((2, chunk, NUM_EXPERTS), jnp.float32),
            pltpu.VMEM((tpw * TOP_K,), jnp.float32),
            pltpu.VMEM((tpw * TOP_K,), jnp.int32),
            pltpu.SemaphoreType.DMA((2,)),
            pltpu.SemaphoreType.DMA((2,)),
            pltpu.SemaphoreType.DMA,
            pltpu.SemaphoreType.DMA,
        ],
    )
    def run(
        logits_hbm, scores_hbm, weights_hbm, inds_hbm,
        lg_v, sc_v, w_v, i_v, sem_in, sem_sc, sem_w, sem_i,
    ):
        wid = lax.axis_index("s") * info.num_cores + lax.axis_index("c")
        base = wid * tpw
        lanes = lax.iota(jnp.int32, NUM_EXPERTS)

        def max_tree(v):
            for s in (1, 2, 4, 8):
                v = jnp.maximum(v, v.at[lanes ^ s].get(mode="promise_in_bounds"))
            return v

        def start_in(c):
            return pltpu.async_copy(
                logits_hbm.at[pl.ds(base + c * chunk, chunk)],
                lg_v.at[c % 2],
                sem_in.at[c % 2],
            )

        in_dma = {0: start_in(0)}
        out_dmas = {}
        for c in range(nchunks):
            if c + 1 < nchunks:
                in_dma[c + 1] = start_in(c + 1)
            in_dma.pop(c).wait()
            if c >= 2:
                for h in out_dmas.pop(c - 2):
                    h.wait()
            b = c % 2
            lg_b, sc_b = lg_v.at[b], sc_v.at[b]
            toff = c * chunk

            @plsc.parallel_loop(0, chunk, unroll=8)
            def body(t):
                row = lg_b[t]
                m1 = max_tree(row)
                i1 = plsc.all_reduce_ffs(row == m1)
                sel1 = lanes == i1
                row2 = jnp.where(sel1, -jnp.inf, row)
                m2 = max_tree(row2)
                i2 = plsc.all_reduce_ffs(row2 == m2)
                sel2 = lanes == i2
                chosen = sel1 | sel2
                sig = 1.0 / (1.0 + jnp.exp(-row))
                sc_b[t] = jnp.where(chosen, sig, 0.0)
                flat = jnp.where(sel1, TOP_K * (toff + t), TOP_K * (toff + t) + 1)
                plsc.store_scatter(w_v, [flat], sig, mask=chosen)
                plsc.store_scatter(i_v, [flat], lanes, mask=chosen)

            cs = pl.ds(base + c * chunk, chunk)
            out_dmas[c] = (
                pltpu.async_copy(sc_b, scores_hbm.at[cs], sem_sc.at[b]),
            )
        fs = pl.ds(base * TOP_K, tpw * TOP_K)
        out_dmas[nchunks] = (
            pltpu.async_copy(w_v, weights_hbm.at[fs], sem_w),
            pltpu.async_copy(i_v, inds_hbm.at[fs], sem_i),
        )
        for hs in out_dmas.values():
            for h in hs:
                h.wait()

    return run(logits)


def kernel(x, W):
    xf = x.reshape(-1, x.shape[-1])
    n = xf.shape[0]
    logits = _logits(xf, W)
    scores, weights_flat, inds_flat = _router(logits, n)
    return (
        scores,
        weights_flat.reshape(n, TOP_K),
        inds_flat.reshape(n, TOP_K),
    )


# speedup vs baseline: 1.1462x; 1.1462x over previous
import jax
import jax.numpy as jnp
from jax import lax
from jax.experimental import pallas as pl
from jax.experimental.pallas import tpu as pltpu
from jax.experimental.pallas import tpu_sc as plsc

HIDDEN = 2048
NUM_EXPERTS = 16
TOP_K = 2


def _logits_body(x_ref, w_ref, out_ref):
    out_ref[...] = lax.dot_general(
        x_ref[...],
        w_ref[...],
        dimension_numbers=(((1,), (1,)), ((), ())),
        preferred_element_type=jnp.float32,
    )


def _logits(xf, W):
    n = xf.shape[0]
    bt = 1024
    return pl.pallas_call(
        _logits_body,
        grid=(n // bt,),
        in_specs=[
            pl.BlockSpec((bt, HIDDEN), lambda i: (i, 0)),
            pl.BlockSpec((NUM_EXPERTS, HIDDEN), lambda i: (0, 0)),
        ],
        out_specs=pl.BlockSpec((bt, NUM_EXPERTS), lambda i: (i, 0)),
        out_shape=jax.ShapeDtypeStruct((n, NUM_EXPERTS), jnp.float32),
        compiler_params=pltpu.CompilerParams(vmem_limit_bytes=100 * 2**20),
    )(xf, W)


def _router(logits, n):
    info = plsc.get_sparse_core_info()
    nw = info.num_cores * info.num_subcores
    tpw = n // nw
    chunk = 128
    mesh = plsc.VectorSubcoreMesh(core_axis_name="c", subcore_axis_name="s")

    nchunks = tpw // chunk

    @pl.kernel(
        out_type=(
            jax.ShapeDtypeStruct((n, NUM_EXPERTS), jnp.float32),
            jax.ShapeDtypeStruct((n, TOP_K), jnp.float32),
            jax.ShapeDtypeStruct((n, TOP_K), jnp.int32),
        ),
        mesh=mesh,
        compiler_params=pltpu.CompilerParams(needs_layout_passes=False),
        scratch_types=[
            pltpu.VMEM((2, chunk, NUM_EXPERTS), jnp.float32),
            pltpu.VMEM((2, chunk, NUM_EXPERTS), jnp.float32),
            pltpu.VMEM((chunk, TOP_K), jnp.float32),
            pltpu.VMEM((chunk, TOP_K), jnp.int32),
            pltpu.SemaphoreType.DMA((2,)),
            pltpu.SemaphoreType.DMA((2,)),
            pltpu.SemaphoreType.DMA,
            pltpu.SemaphoreType.DMA,
        ],
    )
    def run(
        logits_hbm, scores_hbm, weights_hbm, inds_hbm,
        lg_v, sc_v, w_v, i_v, sem_in, sem_sc, sem_w, sem_i,
    ):
        wid = lax.axis_index("s") * info.num_cores + lax.axis_index("c")
        base = wid * tpw
        lanes = lax.iota(jnp.int32, NUM_EXPERTS)

        def max_tree(v):
            for s in (1, 2, 4, 8):
                v = jnp.maximum(v, v.at[lanes ^ s].get(mode="promise_in_bounds"))
            return v

        def start_in(c):
            return pltpu.async_copy(
                logits_hbm.at[pl.ds(base + c * chunk, chunk)],
                lg_v.at[c % 2],
                sem_in.at[c % 2],
            )

        in_dma = {0: start_in(0)}
        out_dmas = {}
        for c in range(nchunks):
            if c + 1 < nchunks:
                in_dma[c + 1] = start_in(c + 1)
            in_dma.pop(c).wait()
            if c >= 2:
                for h in out_dmas.pop(c - 2):
                    h.wait()
            if c >= 1:
                for h in out_dmas.pop(-c):
                    h.wait()
            b = c % 2
            lg_b, sc_b = lg_v.at[b], sc_v.at[b]

            @plsc.parallel_loop(0, chunk, unroll=8)
            def body(t):
                row = lg_b[t]
                m1 = max_tree(row)
                i1 = plsc.all_reduce_ffs(row == m1)
                sel1 = lanes == i1
                row2 = jnp.where(sel1, -jnp.inf, row)
                m2 = max_tree(row2)
                i2 = plsc.all_reduce_ffs(row2 == m2)
                sel2 = lanes == i2
                chosen = sel1 | sel2
                sig = 1.0 / (1.0 + jnp.exp(-row))
                sc_b[t] = jnp.where(chosen, sig, 0.0)
                rowv = jnp.full((NUM_EXPERTS,), t, jnp.int32)
                colv = jnp.where(sel1, 0, 1)
                plsc.store_scatter(w_v, [rowv, colv], sig, mask=chosen)
                plsc.store_scatter(i_v, [rowv, colv], lanes, mask=chosen)

            cs = pl.ds(base + c * chunk, chunk)
            out_dmas[c] = (
                pltpu.async_copy(sc_b, scores_hbm.at[cs], sem_sc.at[b]),
            )
            out_dmas[-(c + 1)] = (
                pltpu.async_copy(w_v, weights_hbm.at[cs], sem_w),
                pltpu.async_copy(i_v, inds_hbm.at[cs], sem_i),
            )
        for hs in out_dmas.values():
            for h in hs:
                h.wait()

    return run(logits)


def kernel(x, W):
    xf = x.reshape(-1, x.shape[-1])
    n = xf.shape[0]
    logits = _logits(xf, W)
    return _router(logits, n)
